# baseline (speedup 1.0000x reference)
import functools

import jax
import jax.numpy as jnp
from jax import lax
from jax.experimental import pallas as pl
from jax.experimental.pallas import tpu as pltpu
from jax.experimental.pallas import tpu_sc as plsc

D = 128
NC = 2
NS = 16
NW = NC * NS
CH = 128
K = 4


@functools.partial(jax.jit, static_argnums=())
def _sc_gather(x_r, weight):
    nw, nchunk, ch = x_r.shape
    b_total = nw * nchunk * ch
    mesh = plsc.VectorSubcoreMesh(core_axis_name="c", subcore_axis_name="s")

    @functools.partial(
        pl.kernel,
        mesh=mesh,
        out_type=jax.ShapeDtypeStruct((b_total, D), jnp.float32),
        scratch_types=[
            pltpu.VMEM((nchunk, ch), jnp.int32),
            pltpu.VMEM((K, ch, D), jnp.float32),
            pltpu.SemaphoreType.DMA,
            pltpu.SemaphoreType.DMA,
        ],
    )
    def k(x_hbm, w_hbm, out_hbm, idx_v, buf_v, gsem, ssem):
        wid = lax.axis_index("s") * NC + lax.axis_index("c")
        base = wid * (nchunk * ch)
        pltpu.sync_copy(x_hbm.at[wid], idx_v)

        def start_gather(c, b):
            return pltpu.async_copy(w_hbm.at[idx_v.at[c]], buf_v.at[b], gsem)

        def start_scatter(c, b):
            row0 = base + c * ch
            return pltpu.async_copy(buf_v.at[b], out_hbm.at[pl.ds(row0, ch)], ssem)

        def wait_scatter(c, b):
            row0 = base + c * ch
            pltpu.make_async_copy(
                buf_v.at[b], out_hbm.at[pl.ds(row0, ch)], ssem
            ).wait()

        gathers = [start_gather(b, b) for b in range(K)]
        for b in range(K):
            gathers[b].wait()
            start_scatter(b, b)

        def group(g, carry):
            c0 = g * K
            gathers = []
            for b in range(K):
                wait_scatter(c0 - K + b, b)
                gathers.append(start_gather(c0 + b, b))
            for b in range(K):
                gathers[b].wait()
                start_scatter(c0 + b, b)
            return carry

        lax.fori_loop(1, nchunk // K, group, 0, unroll=False)

        for b in range(K):
            wait_scatter(nchunk - K + b, b)

    return k(x_r, weight)


def kernel(x, weight):
    batch, fields = x.shape
    b_total = batch * fields
    assert b_total % (NW * CH * K) == 0
    nchunk = b_total // (NW * CH)
    x_r = x.reshape(NW, nchunk, CH).astype(jnp.int32)
    out = _sc_gather(x_r, weight)
    return out.reshape(batch, fields, D)


# speedup vs baseline: 1.7116x; 1.7116x over previous
import functools

import jax
import jax.numpy as jnp
from jax import lax
from jax.experimental import pallas as pl
from jax.experimental.pallas import tpu as pltpu
from jax.experimental.pallas import tpu_sc as plsc

D = 128
NC = 2
NS = 16
NW = NC * NS
R = 4
K = 4
SEC = 128


def _sc_gather(x, weight):
    batch, fields = x.shape
    nb_per_w = batch // NW
    nsec = nb_per_w // SEC
    cps = SEC // R
    mesh = plsc.VectorSubcoreMesh(core_axis_name="c", subcore_axis_name="s")

    @functools.partial(
        pl.kernel,
        mesh=mesh,
        out_type=jax.ShapeDtypeStruct((batch, fields, D), jnp.float32),
        scratch_types=[
            pltpu.VMEM((SEC, fields), jnp.int32),
            pltpu.VMEM((K, R, fields, D), jnp.float32),
            pltpu.SemaphoreType.DMA,
            pltpu.SemaphoreType.DMA,
        ],
    )
    def k(x_hbm, w_hbm, out_hbm, idx_v, buf_v, gsem, ssem):
        wid = lax.axis_index("s") * NC + lax.axis_index("c")
        base = wid * nb_per_w

        def start_gathers(sec_row0, c, b):
            return [
                pltpu.async_copy(
                    w_hbm.at[idx_v.at[c * R + r]], buf_v.at[b, r], gsem
                )
                for r in range(R)
            ]

        def start_scatter(sec_row0, c, b):
            row0 = sec_row0 + c * R
            return pltpu.async_copy(
                buf_v.at[b], out_hbm.at[pl.ds(row0, R)], ssem
            )

        def wait_scatter(sec_row0, c, b):
            row0 = sec_row0 + c * R
            pltpu.make_async_copy(
                buf_v.at[b], out_hbm.at[pl.ds(row0, R)], ssem
            ).wait()

        def section(s, carry):
            sec_row0 = base + s * SEC
            pltpu.sync_copy(x_hbm.at[pl.ds(sec_row0, SEC)], idx_v)

            gathers = [start_gathers(sec_row0, b, b) for b in range(K)]
            for b in range(K):
                for g in gathers[b]:
                    g.wait()
                start_scatter(sec_row0, b, b)

            def group(g, carry2):
                c0 = g * K
                gathers = []
                for b in range(K):
                    wait_scatter(sec_row0, c0 - K + b, b)
                    gathers.append(start_gathers(sec_row0, c0 + b, b))
                for b in range(K):
                    for gg in gathers[b]:
                        gg.wait()
                    start_scatter(sec_row0, c0 + b, b)
                return carry2

            lax.fori_loop(1, cps // K, group, 0, unroll=False)

            for b in range(K):
                wait_scatter(sec_row0, cps - K + b, b)
            return carry

        lax.fori_loop(0, nsec, section, 0, unroll=False)

    return k(x, weight)


def kernel(x, weight):
    batch, fields = x.shape
    assert batch % (NW * SEC) == 0 and SEC % (R * K) == 0
    return _sc_gather(x.astype(jnp.int32), weight)
